# baseline (speedup 1.0000x reference)
import functools

import jax
import jax.numpy as jnp
from jax import lax
from jax.experimental import pallas as pl
from jax.experimental.pallas import tpu as pltpu
from jax.experimental.pallas import tpu_sc as plsc

_N = 10000
_E = 320000
_D = 128
_NC = 2
_NS = 16
_CH = 80
_NB = 3


def _make_sc_agg():
    mesh = plsc.VectorSubcoreMesh(core_axis_name="c", subcore_axis_name="s")
    n_workers = _NC * _NS
    epw = _E // n_workers
    n_chunks = epw // _CH

    rpt0 = 624
    rpt1 = _N - (_NS - 1) * rpt0

    @functools.partial(
        pl.kernel,
        mesh=mesh,
        out_type=jax.ShapeDtypeStruct((_NC, _N, _D), jnp.float32),
        scratch_types=[
            pltpu.VMEM((2 * epw,), jnp.int32),
            pltpu.VMEM((_CH,), jnp.int32),
            pltpu.VMEM((_NB, _CH, _D), jnp.float32),
            pltpu.VMEM_SHARED((_N, _D), jnp.float32),
            pltpu.SemaphoreType.DMA,
            pltpu.SemaphoreType.DMA,
        ] + [pltpu.SemaphoreType.DMA] * _NB,
    )
    def sc_agg(x_hbm, ei_hbm, out_hbm,
               idx_v, dstc, rows, agg_sh, zsem, isem, *gsems):
        gsems = list(gsems)
        cid = lax.axis_index("c")
        sid = lax.axis_index("s")
        wid = sid * _NC + cid
        last = sid == _NS - 1
        r0 = sid * rpt0

        zv = jnp.zeros((16,), jnp.float32)
        for r in range(16):
            for l in range(_D // 16):
                rows[0, r, pl.ds(16 * l, 16)] = zv
        zstrip = rows.at[0, pl.ds(0, 16)]

        base = wid * epw
        pltpu.async_copy(ei_hbm.at[pl.ds(base, epw)],
                         idx_v.at[pl.ds(0, epw)], isem)
        pltpu.async_copy(ei_hbm.at[pl.ds(_E + base, epw)],
                         idx_v.at[pl.ds(epw, epw)], isem)

        def init(start, nrows):
            @pl.when(cid == 0)
            def _():
                pltpu.async_copy(x_hbm.at[pl.ds(start, nrows)],
                                 agg_sh.at[pl.ds(start, nrows)], zsem)
                pltpu.make_async_copy(x_hbm.at[pl.ds(start, nrows)],
                                      agg_sh.at[pl.ds(start, nrows)],
                                      zsem).wait()

            @pl.when(cid == 1)
            def _():
                for j in range(nrows // 16):
                    pltpu.async_copy(zstrip,
                                     agg_sh.at[pl.ds(start + 16 * j, 16)],
                                     zsem)
                for j in range(nrows // 16):
                    pltpu.make_async_copy(zstrip, agg_sh.at[pl.ds(0, 16)],
                                          zsem).wait()

        @pl.when(~last)
        def _():
            init(r0, rpt0)

        @pl.when(last)
        def _():
            init((_NS - 1) * rpt0, rpt1)

        pltpu.make_async_copy(ei_hbm.at[pl.ds(base, epw)],
                              idx_v.at[pl.ds(0, epw)], isem).wait()
        pltpu.make_async_copy(ei_hbm.at[pl.ds(base, epw)],
                              idx_v.at[pl.ds(0, epw)], isem).wait()
        plsc.subcore_barrier()

        def sl(c):
            return pl.ds(pl.multiple_of(c * _CH, _CH), _CH)

        def copy_dst(c):
            off = pl.multiple_of(epw + c * _CH, _CH)
            for j in range(_CH // 16):
                dstc[pl.ds(16 * j, 16)] = idx_v[pl.ds(off + 16 * j, 16)]

        def issue(c, k):
            pltpu.async_copy(x_hbm.at[idx_v.at[sl(c)]], rows.at[k], gsems[k])

        def step(c, k, issue_next):
            pltpu.make_async_copy(x_hbm.at[idx_v.at[sl(0)]], rows.at[k],
                                  gsems[k]).wait()
            copy_dst(c)
            pltpu.sync_copy(rows.at[k], agg_sh.at[dstc], add=True)
            if issue_next:
                cond = c + _NB < n_chunks
                if isinstance(cond, bool):
                    if cond:
                        issue(c + _NB, k)
                else:
                    @pl.when(cond)
                    def _():
                        issue(c + _NB, k)

        for k in range(_NB):
            issue(k, k)

        UNROLL = 24

        def body(i, carry):
            c0 = i * UNROLL
            for t in range(UNROLL):
                step(c0 + t, t % _NB, True)
            return carry

        lax.fori_loop(0, n_chunks // UNROLL, body, 0)
        ntail = n_chunks - UNROLL * (n_chunks // UNROLL)
        for t in range(ntail):
            step(n_chunks - ntail + t, t % _NB, True)
        plsc.subcore_barrier()

        @pl.when(~last)
        def _():
            pltpu.sync_copy(agg_sh.at[pl.ds(r0, rpt0)],
                            out_hbm.at[cid, pl.ds(r0, rpt0)])

        @pl.when(last)
        def _():
            pltpu.sync_copy(agg_sh.at[pl.ds((_NS - 1) * rpt0, rpt1)],
                            out_hbm.at[cid, pl.ds((_NS - 1) * rpt0, rpt1)])

    return sc_agg


_sc_agg = _make_sc_agg()

_BLK = 2000


def _mlp_body(a_ref, w1_ref, b1_ref, w2_ref, b2_ref, o_ref):
    h = a_ref[0] + a_ref[1]
    h = jnp.dot(h, w1_ref[...], preferred_element_type=jnp.float32) + b1_ref[...]
    h = jnp.maximum(h, 0.01 * h)
    h = jnp.dot(h, w2_ref[...], preferred_element_type=jnp.float32) + b2_ref[...]
    o_ref[...] = jnp.maximum(h, 0.01 * h)


def _tc_mlp(agg2, W1, b1, W2, b2):
    return pl.pallas_call(
        _mlp_body,
        grid=(_N // _BLK,),
        in_specs=[
            pl.BlockSpec((_NC, _BLK, _D), lambda i: (0, i, 0)),
            pl.BlockSpec((_D, _D), lambda i: (0, 0)),
            pl.BlockSpec((1, _D), lambda i: (0, 0)),
            pl.BlockSpec((_D, _D), lambda i: (0, 0)),
            pl.BlockSpec((1, _D), lambda i: (0, 0)),
        ],
        out_specs=pl.BlockSpec((_BLK, _D), lambda i: (i, 0)),
        out_shape=jax.ShapeDtypeStruct((_N, _D), jnp.float32),
    )(agg2, W1, b1.reshape(1, _D), W2, b2.reshape(1, _D))


def kernel(x, edge_index, W1, b1, W2, b2):
    agg2 = _sc_agg(x, edge_index.reshape(2 * _E))
    return _tc_mlp(agg2, W1, b1, W2, b2)


# speedup vs baseline: 1.0029x; 1.0029x over previous
import functools

import jax
import jax.numpy as jnp
from jax import lax
from jax.experimental import pallas as pl
from jax.experimental.pallas import tpu as pltpu
from jax.experimental.pallas import tpu_sc as plsc

_N = 10000
_E = 320000
_D = 128
_NC = 2
_NS = 16
_CH = 80
_NB = 3


def _make_sc_agg():
    mesh = plsc.VectorSubcoreMesh(core_axis_name="c", subcore_axis_name="s")
    n_workers = _NC * _NS
    epw = _E // n_workers
    n_chunks = epw // _CH

    rpt0 = 624
    rpt1 = _N - (_NS - 1) * rpt0

    @functools.partial(
        pl.kernel,
        mesh=mesh,
        out_type=jax.ShapeDtypeStruct((_NC, _N, _D), jnp.float32),
        scratch_types=[
            pltpu.VMEM((2 * epw,), jnp.int32),
            pltpu.VMEM((_CH,), jnp.int32),
            pltpu.VMEM((_NB, _CH, _D), jnp.float32),
            pltpu.VMEM_SHARED((_N, _D), jnp.float32),
            pltpu.SemaphoreType.DMA,
            pltpu.SemaphoreType.DMA,
        ] + [pltpu.SemaphoreType.DMA] * _NB,
    )
    def sc_agg(x_hbm, ei_hbm, out_hbm,
               idx_v, dstc, rows, agg_sh, zsem, isem, *gsems):
        gsems = list(gsems)
        cid = lax.axis_index("c")
        sid = lax.axis_index("s")
        wid = sid * _NC + cid
        last = sid == _NS - 1
        r0 = sid * rpt0

        zv = jnp.zeros((16,), jnp.float32)
        for r in range(16):
            for l in range(_D // 16):
                rows[0, r, pl.ds(16 * l, 16)] = zv
        zstrip = rows.at[0, pl.ds(0, 16)]

        base = wid * epw
        pltpu.async_copy(ei_hbm.at[pl.ds(base, epw)],
                         idx_v.at[pl.ds(0, epw)], isem)
        pltpu.async_copy(ei_hbm.at[pl.ds(_E + base, epw)],
                         idx_v.at[pl.ds(epw, epw)], isem)

        def init(start, nrows):
            @pl.when(cid == 0)
            def _():
                pltpu.async_copy(x_hbm.at[pl.ds(start, nrows)],
                                 agg_sh.at[pl.ds(start, nrows)], zsem)
                pltpu.make_async_copy(x_hbm.at[pl.ds(start, nrows)],
                                      agg_sh.at[pl.ds(start, nrows)],
                                      zsem).wait()

            @pl.when(cid == 1)
            def _():
                for j in range(nrows // 16):
                    pltpu.async_copy(zstrip,
                                     agg_sh.at[pl.ds(start + 16 * j, 16)],
                                     zsem)
                for j in range(nrows // 16):
                    pltpu.make_async_copy(zstrip, agg_sh.at[pl.ds(0, 16)],
                                          zsem).wait()

        @pl.when(~last)
        def _():
            init(r0, rpt0)

        @pl.when(last)
        def _():
            init((_NS - 1) * rpt0, rpt1)

        pltpu.make_async_copy(ei_hbm.at[pl.ds(base, epw)],
                              idx_v.at[pl.ds(0, epw)], isem).wait()
        pltpu.make_async_copy(ei_hbm.at[pl.ds(base, epw)],
                              idx_v.at[pl.ds(0, epw)], isem).wait()
        plsc.subcore_barrier()

        def sl(c):
            return pl.ds(pl.multiple_of(c * _CH, _CH), _CH)

        def copy_dst(c):
            off = pl.multiple_of(epw + c * _CH, _CH)
            for j in range(_CH // 16):
                dstc[pl.ds(16 * j, 16)] = idx_v[pl.ds(off + 16 * j, 16)]

        def issue(c, k):
            pltpu.async_copy(x_hbm.at[idx_v.at[sl(c)]], rows.at[k], gsems[k])

        def step(c, k, issue_next):
            pltpu.make_async_copy(x_hbm.at[idx_v.at[sl(0)]], rows.at[k],
                                  gsems[k]).wait()
            copy_dst(c)
            pltpu.sync_copy(rows.at[k], agg_sh.at[dstc], add=True)
            if issue_next:
                @pl.when(c + _NB < n_chunks)
                def _():
                    issue(c + _NB, k)

        for k in range(_NB):
            issue(k, k)

        def body(i, carry):
            c0 = i * _NB
            for k in range(_NB):
                step(c0 + k, k, True)
            return carry

        lax.fori_loop(0, n_chunks // _NB, body, 0)
        ntail = n_chunks - _NB * (n_chunks // _NB)
        for t in range(ntail):
            step(n_chunks - ntail + t, t, False)
        plsc.subcore_barrier()

        @pl.when(~last)
        def _():
            pltpu.sync_copy(agg_sh.at[pl.ds(r0, rpt0)],
                            out_hbm.at[cid, pl.ds(r0, rpt0)])

        @pl.when(last)
        def _():
            pltpu.sync_copy(agg_sh.at[pl.ds((_NS - 1) * rpt0, rpt1)],
                            out_hbm.at[cid, pl.ds((_NS - 1) * rpt0, rpt1)])

    return sc_agg


_sc_agg = _make_sc_agg()

_BLK = 2000


def _mlp_body(a_ref, w1_ref, b1_ref, w2_ref, b2_ref, o_ref):
    h = a_ref[0] + a_ref[1]
    h = jnp.dot(h, w1_ref[...], preferred_element_type=jnp.float32) + b1_ref[...]
    h = jnp.maximum(h, 0.01 * h)
    h = jnp.dot(h, w2_ref[...], preferred_element_type=jnp.float32) + b2_ref[...]
    o_ref[...] = jnp.maximum(h, 0.01 * h)


def _tc_mlp(agg2, W1, b1, W2, b2):
    return pl.pallas_call(
        _mlp_body,
        grid=(_N // _BLK,),
        in_specs=[
            pl.BlockSpec((_NC, _BLK, _D), lambda i: (0, i, 0)),
            pl.BlockSpec((_D, _D), lambda i: (0, 0)),
            pl.BlockSpec((1, _D), lambda i: (0, 0)),
            pl.BlockSpec((_D, _D), lambda i: (0, 0)),
            pl.BlockSpec((1, _D), lambda i: (0, 0)),
        ],
        out_specs=pl.BlockSpec((_BLK, _D), lambda i: (i, 0)),
        out_shape=jax.ShapeDtypeStruct((_N, _D), jnp.float32),
    )(agg2, W1, b1.reshape(1, _D), W2, b2.reshape(1, _D))


def kernel(x, edge_index, W1, b1, W2, b2):
    agg2 = _sc_agg(x, edge_index.reshape(2 * _E))
    return _tc_mlp(agg2, W1, b1, W2, b2)
